# baseline (speedup 1.0000x reference)
import functools

import jax
import jax.numpy as jnp
from jax import lax
from jax.experimental import pallas as pl
from jax.experimental.pallas import tpu as pltpu
from jax.experimental.pallas import tpu_sc as plsc

_BATCH = 16384
_RANK = 32
_NC = 2
_NS = 16
_NW = _NC * _NS
_BPW = _BATCH // _NW
_CHUNK = 128
_NCHUNK = _BPW // _CHUNK
_L = 16
_NGROUP = _BPW // _L

_mesh = plsc.VectorSubcoreMesh(core_axis_name="c", subcore_axis_name="s")


@functools.partial(
    pl.kernel,
    mesh=_mesh,
    compiler_params=pltpu.CompilerParams(
        needs_layout_passes=False, use_tc_tiling_on_sc=False),
    out_type=jax.ShapeDtypeStruct((_BATCH,), jnp.float32),
    scratch_types=[
        pltpu.VMEM((_NCHUNK, _CHUNK), jnp.int32),
        pltpu.VMEM((_NCHUNK, _CHUNK), jnp.int32),
        pltpu.VMEM((_BPW, _RANK), jnp.float32),
        pltpu.VMEM((_BPW, _RANK), jnp.float32),
        pltpu.VMEM((_BPW,), jnp.float32),
        pltpu.SemaphoreType.DMA,
    ],
)
def _mfact_body(users_hbm, movies_hbm, z_hbm, w_hbm, out_hbm,
                idx_u, idx_m, rows_z, rows_w, out_v, sem):
    wid = lax.axis_index("s") * _NC + lax.axis_index("c")
    base = wid * _BPW

    for j in range(_NCHUNK):
        pltpu.sync_copy(users_hbm.at[pl.ds(base + j * _CHUNK, _CHUNK)],
                        idx_u.at[j])
        pltpu.sync_copy(movies_hbm.at[pl.ds(base + j * _CHUNK, _CHUNK)],
                        idx_m.at[j])

    copies = []
    for j in range(_NCHUNK):
        copies.append(pltpu.async_copy(
            z_hbm.at[idx_u.at[j]], rows_z.at[pl.ds(j * _CHUNK, _CHUNK)], sem))
        copies.append(pltpu.async_copy(
            w_hbm.at[idx_m.at[j]], rows_w.at[pl.ds(j * _CHUNK, _CHUNK)], sem))
    for c in copies:
        c.wait()

    def group(g, carry):
        row = g * _L + lax.iota(jnp.int32, _L)
        acc = jnp.zeros((_L,), jnp.float32)
        for k in range(_RANK):
            col = jnp.full((_L,), k, jnp.int32)
            zc = plsc.load_gather(rows_z, [row, col])
            wc = plsc.load_gather(rows_w, [row, col])
            acc = acc + zc * wc
        out_v[pl.ds(g * _L, _L)] = acc
        return carry

    lax.fori_loop(0, _NGROUP, group, 0)

    pltpu.sync_copy(out_v, out_hbm.at[pl.ds(base, _BPW)])


def kernel(users, movies, Z, W):
    return _mfact_body(users.astype(jnp.int32), movies.astype(jnp.int32), Z, W)


# speedup vs baseline: 1.4862x; 1.4862x over previous
import functools

import jax
import jax.numpy as jnp
from jax import lax
from jax.experimental import pallas as pl
from jax.experimental.pallas import tpu as pltpu
from jax.experimental.pallas import tpu_sc as plsc

_BATCH = 16384
_RANK = 32
_NU = 1000000
_NM = 100000
_STRIPE = 32768
_CW = 2048
_NCHUNK = _STRIPE // _CW
_SEG = 768
_SORT = 1024
_NSEG = -(-_BATCH // _SEG)
_L = 16
_TRASH = _BATCH
_ZMAIN = 999936

_mesh = plsc.VectorSubcoreMesh(core_axis_name="c", subcore_axis_name="s")


def _scalar(v):
    return v[0]


@functools.partial(
    pl.kernel,
    mesh=_mesh,
    compiler_params=pltpu.CompilerParams(
        needs_layout_passes=False, use_tc_tiling_on_sc=True),
    out_type=jax.ShapeDtypeStruct((2, _BATCH), jnp.float32),
    scratch_types=[
        pltpu.VMEM((_BATCH,), jnp.int32),
        pltpu.VMEM((_BATCH,), jnp.int32),
        pltpu.VMEM((_SEG + _L,), jnp.int32),
        pltpu.VMEM((_SEG + _L,), jnp.int32),
        pltpu.VMEM((_SORT + _L,), jnp.int32),
        pltpu.VMEM((_SORT + _L,), jnp.int32),
        pltpu.VMEM((_SORT + _L,), jnp.int32),
        pltpu.VMEM((_L,), jnp.int32),
        pltpu.VMEM((_L,), jnp.int32),
        pltpu.VMEM((2, 8, _CW), jnp.float32),
        pltpu.VMEM((_RANK, _SORT), jnp.float32),
        pltpu.VMEM((128, 128), jnp.float32),
        pltpu.VMEM((128,), jnp.float32),
        pltpu.VMEM((8, 128), jnp.int32),
        pltpu.VMEM((1152,), jnp.float32),
        pltpu.VMEM((4, 8, 64), jnp.float32),
        pltpu.VMEM_SHARED((18432,), jnp.float32),
        pltpu.SemaphoreType.DMA,
        pltpu.SemaphoreType.DMA,
        pltpu.SemaphoreType.DMA,
    ],
)
def _mfact_body(users_hbm, movies_hbm, zt3_hbm, w128_hbm, ztail_hbm,
                out_hbm,
                users_v, movies_v, selu, selb, sortu, sortb, sortm,
                choff, chcnt, zring, zselT, wwave, dots, sb2, zbuf,
                ztail_v, acc_sh, sem0, sem1, semw):
    cid = lax.axis_index("c")
    tid = lax.axis_index("s")
    wid = tid * 2 + cid
    u_lo = wid * _STRIPE

    cp_u = pltpu.async_copy(users_hbm, users_v, sem0)
    cp_m = pltpu.async_copy(movies_hbm, movies_v, sem1)
    pltpu.sync_copy(ztail_hbm, ztail_v)
    zv = jnp.zeros((_L,), jnp.float32)
    for g in range(1152 // _L):
        zbuf[pl.ds(g * _L, _L)] = zv
    pltpu.sync_copy(zbuf, acc_sh.at[pl.ds(tid * 1152, 1152)])
    cp_u.wait()
    cp_m.wait()
    plsc.subcore_barrier()

    lanes = lax.iota(jnp.int32, _L)

    def count_body(g, tot):
        u = users_v[pl.ds(g * _L, _L)]
        msk = (u >= u_lo) & (u < u_lo + _STRIPE)
        return tot + _scalar(plsc.all_reduce_population_count(msk))
    n_total = lax.fori_loop(0, _BATCH // _L, count_body, jnp.int32(0))

    def segment(p, carry):
        @pl.when(p * _SEG < n_total)
        def _():
            win_lo = p * _SEG
            ufill = jnp.full((_L,), u_lo, jnp.int32)
            bfill = jnp.full((_L,), _TRASH, jnp.int32)
            mfill = jnp.zeros((_L,), jnp.int32)
            for g in range(_SEG // _L):
                selu[pl.ds(g * _L, _L)] = ufill
                selb[pl.ds(g * _L, _L)] = bfill
            for g in range(_SORT // _L):
                sortu[pl.ds(g * _L, _L)] = ufill
                sortb[pl.ds(g * _L, _L)] = bfill
                sortm[pl.ds(g * _L, _L)] = mfill

            def sel_body(g, c):
                t, cnt = c
                u = users_v[pl.ds(g * _L, _L)]
                msk = (u >= u_lo) & (u < u_lo + _STRIPE)
                pos = t + plsc.cumsum(msk.astype(jnp.int32)) - 1
                m2 = msk & (pos >= win_lo) & (pos < win_lo + _SEG)
                b = g * _L + lanes
                plsc.store_compressed(selu.at[pl.ds(cnt, _L)], u, mask=m2)
                plsc.store_compressed(selb.at[pl.ds(cnt, _L)], b, mask=m2)
                t = t + _scalar(plsc.all_reduce_population_count(msk))
                cnt = cnt + _scalar(plsc.all_reduce_population_count(m2))
                return t, cnt
            _, n_seg = lax.fori_loop(0, _BATCH // _L, sel_body,
                                     (jnp.int32(0), jnp.int32(0)))
            n_grp = (n_seg + _L - 1) // _L

            def split_chunk(k, cur):
                def sweep(g, c2):
                    u = selu[pl.ds(g * _L, _L)]
                    b = selb[pl.ds(g * _L, _L)]
                    msk = lax.shift_right_logical(u - u_lo, 11) == k
                    plsc.store_compressed(sortu.at[pl.ds(c2, _L)], u,
                                          mask=msk)
                    plsc.store_compressed(sortb.at[pl.ds(c2, _L)], b,
                                          mask=msk)
                    return c2 + _scalar(
                        plsc.all_reduce_population_count(msk))
                nxt = lax.fori_loop(0, n_grp, sweep, cur)
                nxt = ((nxt + _L - 1) // _L) * _L
                kv = jnp.full((_L,), k, jnp.int32)
                choff[...] = jnp.where(lanes == kv, cur, choff[...])
                chcnt[...] = jnp.where(lanes == kv, nxt - cur, chcnt[...])
                return nxt
            tot_sorted = lax.fori_loop(0, _NCHUNK, split_chunk,
                                       jnp.int32(0))

            def m_body(g, c):
                b = sortb[pl.ds(g * _L, _L)]
                bs = jnp.minimum(b, _BATCH - 1)
                sortm[pl.ds(g * _L, _L)] = plsc.load_gather(movies_v, [bs])
                return c
            lax.fori_loop(0, tot_sorted // _L, m_body, 0)
            for g in range(_SORT // _L):
                sb2[g // 8, pl.ds((g % 8) * _L, _L)] = \
                    sortb[pl.ds(g * _L, _L)]

            offs = choff[...]
            cnts = chcnt[...]

            def chunk_start(c):
                c0 = jnp.minimum(u_lo + c * _CW, _ZMAIN - _CW)
                return pl.multiple_of(c0, 128)

            cp = pltpu.async_copy(
                zt3_hbm.at[0, pl.ds(0, 8), pl.ds(chunk_start(0), _CW)],
                zring.at[0], sem0)
            for step in range(4 * _NCHUNK):
                a, c = step // _NCHUNK, step % _NCHUNK
                slot = step % 2
                cp.wait()
                if step + 1 < 4 * _NCHUNK:
                    a2, c2 = (step + 1) // _NCHUNK, (step + 1) % _NCHUNK
                    cp = pltpu.async_copy(
                        zt3_hbm.at[a2, pl.ds(0, 8),
                                   pl.ds(chunk_start(c2), _CW)],
                        zring.at[1 - slot],
                        sem1 if (1 - slot) else sem0)
                c0 = chunk_start(c)
                off_c = offs[c]
                grp_c = (cnts[c] + _L - 1) // _L
                has_tail = (c == 8)

                def z_grp(i, carry, a=a, slot=slot, off_c=off_c, c0=c0,
                          has_tail=has_tail):
                    e0 = off_c + i * _L
                    u = sortu[pl.ds(e0, _L)]
                    ul = jnp.clip(u - c0, 0, _CW - 1)
                    if has_tail:
                        tmask = u >= _ZMAIN
                        ut = jnp.clip(u - _ZMAIN, 0, 63)
                        af = jnp.full((_L,), a, jnp.int32)
                    for rr in range(8):
                        row = jnp.full((_L,), rr, jnp.int32)
                        v = plsc.load_gather(zring.at[slot], [row, ul])
                        if has_tail:
                            vt = plsc.load_gather(ztail_v, [af, row, ut])
                            v = jnp.where(tmask, vt, v)
                        zselT[8 * a + rr, pl.ds(e0, _L)] = v
                    return carry
                lax.fori_loop(0, grp_c, z_grp, 0)

            def wave(q, carry):
                pltpu.async_copy(
                    w128_hbm.at[sortm.at[pl.ds(q * 128, 128)]],
                    wwave, semw).wait()

                def dot_grp(i, c2):
                    e0 = q * 128 + i * _L
                    erow = i * _L + lanes
                    s = jnp.zeros((_L,), jnp.float32)
                    for r in range(_RANK):
                        col = jnp.full((_L,), r, jnp.int32)
                        wv = plsc.load_gather(wwave, [erow, col])
                        s = s + zselT[r, pl.ds(e0, _L)] * wv
                    dots[pl.ds(i * _L, _L)] = s
                    return c2
                lax.fori_loop(0, 8, dot_grp, 0)
                pltpu.sync_copy(dots, acc_sh.at[sb2.at[q]], add=True)
                return carry
            lax.fori_loop(0, (tot_sorted + 127) // 128, wave, 0)
        return carry

    lax.fori_loop(0, _NSEG, segment, 0)

    plsc.subcore_barrier()
    pltpu.sync_copy(acc_sh.at[pl.ds(tid * 1024, 1024)],
                    out_hbm.at[cid, pl.ds(tid * 1024, 1024)])


def kernel(users, movies, Z, W):
    zt3 = Z.T.reshape(4, 8, _NU)
    w128 = jnp.pad(W, ((0, 0), (0, 128 - _RANK)))
    ztail = Z[_ZMAIN:].T.reshape(4, 8, _NU - _ZMAIN)
    out = _mfact_body(users.astype(jnp.int32), movies.astype(jnp.int32),
                      zt3, w128, ztail)
    return out[0] + out[1]


# speedup vs baseline: 1.5856x; 1.0669x over previous
import functools

import jax
import jax.numpy as jnp
from jax import lax
from jax.experimental import pallas as pl
from jax.experimental.pallas import tpu as pltpu
from jax.experimental.pallas import tpu_sc as plsc

_BATCH = 16384
_RANK = 32
_NU = 1000000
_NM = 100000
_STRIPE = 32768
_CW = 2048
_NCHUNK = _STRIPE // _CW
_SEG = 768
_SORT = 1024
_NSEG = -(-_BATCH // _SEG)
_L = 16
_TRASH = _BATCH
_ZMAIN = 999936

_mesh = plsc.VectorSubcoreMesh(core_axis_name="c", subcore_axis_name="s")


def _scalar(v):
    return v[0]


@functools.partial(
    pl.kernel,
    mesh=_mesh,
    compiler_params=pltpu.CompilerParams(
        needs_layout_passes=False, use_tc_tiling_on_sc=True),
    out_type=jax.ShapeDtypeStruct((2, _BATCH), jnp.float32),
    scratch_types=[
        pltpu.VMEM((_BATCH,), jnp.int32),
        pltpu.VMEM((_BATCH,), jnp.int32),
        pltpu.VMEM((_SEG + _L,), jnp.int32),
        pltpu.VMEM((_SEG + _L,), jnp.int32),
        pltpu.VMEM((_SORT + _L,), jnp.int32),
        pltpu.VMEM((_SORT + _L,), jnp.int32),
        pltpu.VMEM((_SORT + _L,), jnp.int32),
        pltpu.VMEM((_L,), jnp.int32),
        pltpu.VMEM((_L,), jnp.int32),
        pltpu.VMEM((2, 8, _CW), jnp.float32),
        pltpu.VMEM((_RANK, _SORT), jnp.float32),
        pltpu.VMEM((128, 128), jnp.float32),
        pltpu.VMEM((128,), jnp.float32),
        pltpu.VMEM((8, 128), jnp.int32),
        pltpu.VMEM((1152,), jnp.float32),
        pltpu.VMEM((4, 8, 64), jnp.float32),
        pltpu.VMEM_SHARED((18432,), jnp.float32),
        pltpu.SemaphoreType.DMA,
        pltpu.SemaphoreType.DMA,
        pltpu.SemaphoreType.DMA,
    ],
)
def _mfact_body(users_hbm, movies_hbm, zt3_hbm, w128_hbm, ztail_hbm,
                out_hbm,
                users_v, movies_v, selu, selb, sortu, sortb, sortm,
                choff, chcnt, zring, zselT, wwave, dots, sb2, zbuf,
                ztail_v, acc_sh, sem0, sem1, semw):
    cid = lax.axis_index("c")
    tid = lax.axis_index("s")
    wid = tid * 2 + cid
    u_lo = wid * _STRIPE

    cp_u = pltpu.async_copy(users_hbm, users_v, sem0)
    cp_m = pltpu.async_copy(movies_hbm, movies_v, sem1)
    pltpu.sync_copy(ztail_hbm, ztail_v)
    zv = jnp.zeros((_L,), jnp.float32)
    for g in range(1152 // _L):
        zbuf[pl.ds(g * _L, _L)] = zv
    pltpu.sync_copy(zbuf, acc_sh.at[pl.ds(tid * 1152, 1152)])
    cp_u.wait()
    cp_m.wait()
    plsc.subcore_barrier()

    lanes = lax.iota(jnp.int32, _L)

    def segment(p, n_tot):
        def active(_):
            win_lo = p * _SEG
            ufill = jnp.full((_L,), u_lo, jnp.int32)
            bfill = jnp.full((_L,), _TRASH, jnp.int32)
            mfill = jnp.zeros((_L,), jnp.int32)
            for g in range(_SEG // _L):
                selu[pl.ds(g * _L, _L)] = ufill
                selb[pl.ds(g * _L, _L)] = bfill
            for g in range(_SORT // _L):
                sortu[pl.ds(g * _L, _L)] = ufill
                sortb[pl.ds(g * _L, _L)] = bfill
                sortm[pl.ds(g * _L, _L)] = mfill

            def sel_body(g, c):
                t, cnt = c
                u = users_v[pl.ds(g * _L, _L)]
                msk = (u >= u_lo) & (u < u_lo + _STRIPE)
                pos = t + plsc.cumsum(msk.astype(jnp.int32)) - 1
                m2 = msk & (pos >= win_lo) & (pos < win_lo + _SEG)
                b = g * _L + lanes
                plsc.store_compressed(selu.at[pl.ds(cnt, _L)], u, mask=m2)
                plsc.store_compressed(selb.at[pl.ds(cnt, _L)], b, mask=m2)
                t = t + _scalar(plsc.all_reduce_population_count(msk))
                cnt = cnt + _scalar(plsc.all_reduce_population_count(m2))
                return t, cnt
            t_all, n_seg = lax.fori_loop(0, _BATCH // _L, sel_body,
                                         (jnp.int32(0), jnp.int32(0)))
            n_grp = (n_seg + _L - 1) // _L

            def split_chunk(k, cur):
                def sweep(g, c2):
                    u = selu[pl.ds(g * _L, _L)]
                    b = selb[pl.ds(g * _L, _L)]
                    msk = lax.shift_right_logical(u - u_lo, 11) == k
                    plsc.store_compressed(sortu.at[pl.ds(c2, _L)], u,
                                          mask=msk)
                    plsc.store_compressed(sortb.at[pl.ds(c2, _L)], b,
                                          mask=msk)
                    return c2 + _scalar(
                        plsc.all_reduce_population_count(msk))
                nxt = lax.fori_loop(0, n_grp, sweep, cur)
                nxt = ((nxt + _L - 1) // _L) * _L
                kv = jnp.full((_L,), k, jnp.int32)
                choff[...] = jnp.where(lanes == kv, cur, choff[...])
                chcnt[...] = jnp.where(lanes == kv, nxt - cur, chcnt[...])
                return nxt
            tot_sorted = lax.fori_loop(0, _NCHUNK, split_chunk,
                                       jnp.int32(0))

            def m_body(g, c):
                b = sortb[pl.ds(g * _L, _L)]
                bs = jnp.minimum(b, _BATCH - 1)
                sortm[pl.ds(g * _L, _L)] = plsc.load_gather(movies_v, [bs])
                return c
            lax.fori_loop(0, tot_sorted // _L, m_body, 0)
            for g in range(_SORT // _L):
                sb2[g // 8, pl.ds((g % 8) * _L, _L)] = \
                    sortb[pl.ds(g * _L, _L)]

            offs = choff[...]
            cnts = chcnt[...]

            def chunk_start(c):
                c0 = jnp.minimum(u_lo + c * _CW, _ZMAIN - _CW)
                return pl.multiple_of(c0, 128)

            cp = pltpu.async_copy(
                zt3_hbm.at[0, pl.ds(0, 8), pl.ds(chunk_start(0), _CW)],
                zring.at[0], sem0)
            for step in range(4 * _NCHUNK):
                a, c = step // _NCHUNK, step % _NCHUNK
                slot = step % 2
                cp_prev = cp
                if step + 1 < 4 * _NCHUNK:
                    a2, c2 = (step + 1) // _NCHUNK, (step + 1) % _NCHUNK
                    cp = pltpu.async_copy(
                        zt3_hbm.at[a2, pl.ds(0, 8),
                                   pl.ds(chunk_start(c2), _CW)],
                        zring.at[1 - slot],
                        sem1 if (1 - slot) else sem0)
                cp_prev.wait()
                c0 = chunk_start(c)
                off_c = offs[c]
                grp_c = (cnts[c] + _L - 1) // _L
                has_tail = (c == 8)

                def z_grp(i, carry, a=a, slot=slot, off_c=off_c, c0=c0,
                          has_tail=has_tail):
                    e0 = off_c + i * _L
                    u = sortu[pl.ds(e0, _L)]
                    ul = jnp.clip(u - c0, 0, _CW - 1)
                    if has_tail:
                        tmask = u >= _ZMAIN
                        ut = jnp.clip(u - _ZMAIN, 0, 63)
                        af = jnp.full((_L,), a, jnp.int32)
                    for rr in range(8):
                        row = jnp.full((_L,), rr, jnp.int32)
                        v = plsc.load_gather(zring.at[slot], [row, ul])
                        if has_tail:
                            vt = plsc.load_gather(ztail_v, [af, row, ut])
                            v = jnp.where(tmask, vt, v)
                        zselT[8 * a + rr, pl.ds(e0, _L)] = v
                    return carry
                lax.fori_loop(0, grp_c, z_grp, 0)

            def wave(q, carry):
                pltpu.async_copy(
                    w128_hbm.at[sortm.at[pl.ds(q * 128, 128)]],
                    wwave, semw).wait()

                def dot_grp(i, c2):
                    e0 = q * 128 + i * _L
                    erow = i * _L + lanes
                    s = jnp.zeros((_L,), jnp.float32)
                    for r in range(_RANK):
                        col = jnp.full((_L,), r, jnp.int32)
                        wv = plsc.load_gather(wwave, [erow, col])
                        s = s + zselT[r, pl.ds(e0, _L)] * wv
                    dots[pl.ds(i * _L, _L)] = s
                    return c2
                lax.fori_loop(0, 8, dot_grp, 0)
                pltpu.sync_copy(dots, acc_sh.at[sb2.at[q]], add=True)
                return carry
            lax.fori_loop(0, (tot_sorted + 127) // 128, wave, 0)
            return t_all
        return lax.cond(p * _SEG < n_tot, active, lambda _: n_tot, 0)

    lax.fori_loop(0, _NSEG, segment, jnp.int32(0x7FFFFFFF))

    plsc.subcore_barrier()
    pltpu.sync_copy(acc_sh.at[pl.ds(tid * 1024, 1024)],
                    out_hbm.at[cid, pl.ds(tid * 1024, 1024)])


def kernel(users, movies, Z, W):
    zt3 = Z.T.reshape(4, 8, _NU)
    w128 = jnp.pad(W, ((0, 0), (0, 128 - _RANK)))
    ztail = Z[_ZMAIN:].T.reshape(4, 8, _NU - _ZMAIN)
    out = _mfact_body(users.astype(jnp.int32), movies.astype(jnp.int32),
                      zt3, w128, ztail)
    return out[0] + out[1]
